# baseline (speedup 1.0000x reference)
import functools

import jax
import jax.numpy as jnp
from jax import lax
from jax.experimental import pallas as pl
from jax.experimental.pallas import tpu as pltpu
from jax.experimental.pallas import tpu_sc as plsc

B = 8
NV = 6890
R = 75
MV = 40
NR = R * MV
NRP = 3008
KF = 8
KA = 8

NW = 32
PER_W = NRP // 2
TOT = NW * PER_W

NRG = 15
RT = NRG * MV
NT = R // NRG


def _sc_gather(table, idx):
    mesh = plsc.VectorSubcoreMesh(core_axis_name="c", subcore_axis_name="s")

    @functools.partial(
        pl.kernel,
        out_type=jax.ShapeDtypeStruct((TOT, KF), jnp.float32),
        mesh=mesh,
        scratch_types=[
            pltpu.VMEM((PER_W,), jnp.int32),
            pltpu.VMEM((PER_W, KF), jnp.float32),
            pltpu.SemaphoreType.DMA,
        ],
        compiler_params=pltpu.CompilerParams(use_tc_tiling_on_sc=False),
    )
    def gather_kernel(table_hbm, idx_hbm, out_hbm, idx_v, rows_v, sem):
        wid = lax.axis_index("s") * 2 + lax.axis_index("c")
        base = wid * PER_W
        pltpu.sync_copy(idx_hbm.at[pl.ds(base, PER_W)], idx_v)
        chunks = [(j, 128) for j in range(0, PER_W - 96, 128)]
        chunks.append((PER_W - 96, 96))
        for j, c in chunks:
            pltpu.async_copy(
                table_hbm.at[idx_v.at[pl.ds(j, c)]],
                rows_v.at[pl.ds(j, c)],
                sem,
            )
        for j, c in chunks:
            pltpu.make_async_copy(
                table_hbm.at[idx_v.at[pl.ds(j, c)]],
                rows_v.at[pl.ds(j, c)],
                sem,
            ).wait()
        pltpu.sync_copy(rows_v, out_hbm.at[pl.ds(base, PER_W)])

    return gather_kernel(table, idx)


def _dense_body(g1, g2, g1f, g2f, w1, w2, out, b1aug, b2aug):
    t = pl.program_id(1)

    @pl.when(t == 0)
    def _build_baug():
        for src, dst in ((g1f, b1aug), (g2f, b2aug)):
            coords = src[0].T[:3, :]
            yy = jnp.sum(coords * coords, axis=0, keepdims=True)
            ones = jnp.ones_like(yy)
            zeros = jnp.zeros((KA - 5, NRP), jnp.float32)
            dst[...] = jnp.concatenate([coords, ones, yy, zeros], axis=0)
        out[...] = jnp.zeros_like(out)

    def half(a_ref, baug, w):
        a = a_ref[0]
        ac = a[:, :3]
        xx = jnp.sum(ac * ac, axis=1, keepdims=True)
        ones = jnp.ones_like(xx)
        zeros = jnp.zeros((RT, KA - 5), jnp.float32)
        aaug = jnp.concatenate([-2.0 * ac, xx, ones, zeros], axis=1)
        d2 = jnp.dot(aaug, baug[...], preferred_element_type=jnp.float32)
        d2 = jnp.maximum(d2, 1e-12)
        cmin = jnp.min(d2.reshape(NRG, MV, NRP), axis=1)
        mask = w[0].astype(jnp.float32)
        return jnp.sum(cmin * mask)

    contrib = half(g1, b2aug, w1) + half(g2, b1aug, w2)
    out[...] += contrib * (1.0 / MV)


def _dense_call(g1f, g2f, w1, w2):
    return pl.pallas_call(
        _dense_body,
        grid=(B, NT),
        in_specs=[
            pl.BlockSpec((1, RT, KF), lambda b, t: (b, t, 0)),
            pl.BlockSpec((1, RT, KF), lambda b, t: (b, t, 0)),
            pl.BlockSpec((1, NRP, KF), lambda b, t: (b, 0, 0)),
            pl.BlockSpec((1, NRP, KF), lambda b, t: (b, 0, 0)),
            pl.BlockSpec((1, NRG, NRP), lambda b, t: (b * NT + t, 0, 0)),
            pl.BlockSpec((1, NRG, NRP), lambda b, t: (b * NT + t, 0, 0)),
        ],
        out_specs=pl.BlockSpec((1, 1, 128), lambda b, t: (b, 0, 0)),
        out_shape=jax.ShapeDtypeStruct((B, 1, 128), jnp.float32),
        scratch_shapes=[
            pltpu.VMEM((KA, NRP), jnp.float32),
            pltpu.VMEM((KA, NRP), jnp.float32),
        ],
        compiler_params=pltpu.CompilerParams(
            dimension_semantics=("arbitrary", "arbitrary")),
    )(g1f, g2f, g1f, g2f, w1, w2)


def kernel(v1, v2, cmap, rid_to_vid_list):
    f32 = jnp.float32
    v1 = v1.astype(f32)
    v2 = v2.astype(f32)

    t = jnp.concatenate([v1.reshape(B * NV, 3), v2.reshape(B * NV, 3)], axis=0)
    t = jnp.pad(t, ((0, 0), (0, KF - 3)))

    rid = rid_to_vid_list.reshape(-1).astype(jnp.int32)
    rid = jnp.pad(rid, (0, NRP - NR))
    boff = (jnp.arange(B, dtype=jnp.int32) * NV)[None, :, None]
    soff = (jnp.arange(2, dtype=jnp.int32) * (B * NV))[:, None, None]
    idx = (rid[None, None, :] + boff + soff).reshape(-1)

    g = _sc_gather(t, idx)
    g4 = g.reshape(2, B, NRP, KF)

    wb = (cmap != 0).astype(jnp.int8)
    w1 = jnp.pad(jnp.repeat(wb, MV, axis=2), ((0, 0), (0, 0), (0, NRP - NR)))
    w2 = jnp.pad(jnp.repeat(wb.transpose(0, 2, 1), MV, axis=2),
                 ((0, 0), (0, 0), (0, NRP - NR)))
    w1 = w1.reshape(B * NT, NRG, NRP)
    w2 = w2.reshape(B * NT, NRG, NRP)

    out = _dense_call(g4[0], g4[1], w1, w2)
    return out[:, 0, 0]


# speedup vs baseline: 1.0592x; 1.0592x over previous
import functools

import jax
import jax.numpy as jnp
from jax import lax
from jax.experimental import pallas as pl
from jax.experimental.pallas import tpu as pltpu
from jax.experimental.pallas import tpu_sc as plsc

B = 8
NV = 6890
R = 75
MV = 40
NR = R * MV
NRP = 3008
KF = 8
KA = 8

NW = 32
PER_W = NRP // 2
TOT = NW * PER_W

NRG = 25
RT = NRG * MV
NT = R // NRG


def _sc_gather(table, idx):
    mesh = plsc.VectorSubcoreMesh(core_axis_name="c", subcore_axis_name="s")

    @functools.partial(
        pl.kernel,
        out_type=jax.ShapeDtypeStruct((TOT, KF), jnp.float32),
        mesh=mesh,
        scratch_types=[
            pltpu.VMEM((PER_W,), jnp.int32),
            pltpu.VMEM((PER_W, KF), jnp.float32),
            pltpu.SemaphoreType.DMA,
        ],
        compiler_params=pltpu.CompilerParams(use_tc_tiling_on_sc=False),
    )
    def gather_kernel(table_hbm, idx_hbm, out_hbm, idx_v, rows_v, sem):
        wid = lax.axis_index("s") * 2 + lax.axis_index("c")
        base = wid * PER_W
        pltpu.sync_copy(idx_hbm.at[pl.ds(base, PER_W)], idx_v)
        chunks = [(j, 128) for j in range(0, PER_W - 96, 128)]
        chunks.append((PER_W - 96, 96))
        for j, c in chunks:
            pltpu.async_copy(
                table_hbm.at[idx_v.at[pl.ds(j, c)]],
                rows_v.at[pl.ds(j, c)],
                sem,
            )
        for j, c in chunks:
            pltpu.make_async_copy(
                table_hbm.at[idx_v.at[pl.ds(j, c)]],
                rows_v.at[pl.ds(j, c)],
                sem,
            ).wait()
        pltpu.sync_copy(rows_v, out_hbm.at[pl.ds(base, PER_W)])

    return gather_kernel(table, idx)


def _dense_body(g1, g2, g1f, g2f, cm1, cm2, out, b1aug, b2aug, seg):
    b = pl.program_id(0)
    t = pl.program_id(1)

    @pl.when((b == 0) & (t == 0))
    def _build_seg():
        jr = lax.broadcasted_iota(jnp.int32, (NRP, R), 0) // MV
        hc = lax.broadcasted_iota(jnp.int32, (NRP, R), 1)
        seg[...] = (jr == hc).astype(jnp.float32)

    @pl.when(t == 0)
    def _build_baug():
        for src, dst in ((g1f, b1aug), (g2f, b2aug)):
            coords = src[0].T[:3, :]
            yy = jnp.sum(coords * coords, axis=0, keepdims=True)
            ones = jnp.ones_like(yy)
            zeros = jnp.zeros((KA - 5, NRP), jnp.float32)
            dst[...] = jnp.concatenate([coords, ones, yy, zeros], axis=0)
        out[...] = jnp.zeros_like(out)

    def half(a_ref, baug, cm):
        a = a_ref[0]
        ac = a[:, :3]
        xx = jnp.sum(ac * ac, axis=1, keepdims=True)
        ones = jnp.ones_like(xx)
        zeros = jnp.zeros((RT, KA - 5), jnp.float32)
        aaug = jnp.concatenate([-2.0 * ac, xx, ones, zeros], axis=1)
        d2 = jnp.dot(aaug, baug[...], preferred_element_type=jnp.float32)
        d2 = jnp.maximum(d2, 1e-12)
        cmin = jnp.min(d2.reshape(NRG, MV, NRP), axis=1)
        s = jnp.dot(cmin, seg[...], preferred_element_type=jnp.float32)
        mask = (cm[0] != 0.0).astype(jnp.float32)
        return jnp.sum(s * mask)

    contrib = half(g1, b2aug, cm1) + half(g2, b1aug, cm2)
    out[...] += contrib * (1.0 / MV)


def _dense_call(g1f, g2f, cm1, cm2):
    return pl.pallas_call(
        _dense_body,
        grid=(B, NT),
        in_specs=[
            pl.BlockSpec((1, RT, KF), lambda b, t: (b, t, 0)),
            pl.BlockSpec((1, RT, KF), lambda b, t: (b, t, 0)),
            pl.BlockSpec((1, NRP, KF), lambda b, t: (b, 0, 0)),
            pl.BlockSpec((1, NRP, KF), lambda b, t: (b, 0, 0)),
            pl.BlockSpec((1, NRG, R), lambda b, t: (b * NT + t, 0, 0)),
            pl.BlockSpec((1, NRG, R), lambda b, t: (b * NT + t, 0, 0)),
        ],
        out_specs=pl.BlockSpec((1, 1, 128), lambda b, t: (b, 0, 0)),
        out_shape=jax.ShapeDtypeStruct((B, 1, 128), jnp.float32),
        scratch_shapes=[
            pltpu.VMEM((KA, NRP), jnp.float32),
            pltpu.VMEM((KA, NRP), jnp.float32),
            pltpu.VMEM((NRP, R), jnp.float32),
        ],
        compiler_params=pltpu.CompilerParams(
            dimension_semantics=("arbitrary", "arbitrary")),
    )(g1f, g2f, g1f, g2f, cm1, cm2)


def kernel(v1, v2, cmap, rid_to_vid_list):
    f32 = jnp.float32
    v1 = v1.astype(f32)
    v2 = v2.astype(f32)

    t = jnp.concatenate([v1.reshape(B * NV, 3), v2.reshape(B * NV, 3)], axis=0)
    t = jnp.pad(t, ((0, 0), (0, KF - 3)))

    rid = rid_to_vid_list.reshape(-1).astype(jnp.int32)
    rid = jnp.pad(rid, (0, NRP - NR))
    boff = (jnp.arange(B, dtype=jnp.int32) * NV)[None, :, None]
    soff = (jnp.arange(2, dtype=jnp.int32) * (B * NV))[:, None, None]
    idx = (rid[None, None, :] + boff + soff).reshape(-1)

    g = _sc_gather(t, idx)
    g4 = g.reshape(2, B, NRP, KF)

    cm1 = cmap.astype(f32).reshape(B * NT, NRG, R)
    cm2 = cmap.astype(f32).transpose(0, 2, 1).reshape(B * NT, NRG, R)

    out = _dense_call(g4[0], g4[1], cm1, cm2)
    return out[:, 0, 0]


# speedup vs baseline: 1.2097x; 1.1421x over previous
import functools

import jax
import jax.numpy as jnp
from jax import lax
from jax.experimental import pallas as pl
from jax.experimental.pallas import tpu as pltpu
from jax.experimental.pallas import tpu_sc as plsc

B = 8
NV = 6890
R = 75
RP = 80
MV = 40
NR = R * MV
NRP = 3200
KF = 8
KA = 8

NW = 32
PER_W = NRP // 2
TOT = NW * PER_W

NRG = 16
RT = NRG * MV
NT = RP // NRG


def _sc_gather(table, idx):
    mesh = plsc.VectorSubcoreMesh(core_axis_name="c", subcore_axis_name="s")

    @functools.partial(
        pl.kernel,
        out_type=jax.ShapeDtypeStruct((2, B, KA, NRP), jnp.float32),
        mesh=mesh,
        scratch_types=[
            pltpu.VMEM((PER_W,), jnp.int32),
            pltpu.VMEM((PER_W, KF), jnp.float32),
            pltpu.VMEM((KA, PER_W), jnp.float32),
            pltpu.SemaphoreType.DMA,
        ],
        compiler_params=pltpu.CompilerParams(
            use_tc_tiling_on_sc=False, needs_layout_passes=False),
    )
    def gather_kernel(table_hbm, idx_hbm, out_hbm, idx_v, rows_v, pla_v, sem):
        wid = lax.axis_index("s") * 2 + lax.axis_index("c")
        base = wid * PER_W
        pltpu.sync_copy(idx_hbm.at[pl.ds(base, PER_W)], idx_v)
        chunks = [(j, 128) for j in range(0, PER_W - 64, 128)]
        chunks.append((PER_W - 64, 64))
        for j, c in chunks:
            pltpu.async_copy(
                table_hbm.at[idx_v.at[pl.ds(j, c)]],
                rows_v.at[pl.ds(j, c)],
                sem,
            )
        for j, c in chunks:
            pltpu.make_async_copy(
                table_hbm.at[idx_v.at[pl.ds(j, c)]],
                rows_v.at[pl.ds(j, c)],
                sem,
            ).wait()

        lane = lax.iota(jnp.int32, 16)
        czero = jnp.zeros((16,), jnp.int32)
        fone = jnp.ones((16,), jnp.float32)
        fzero = jnp.zeros((16,), jnp.float32)

        def body(i, carry):
            r0 = i * 16
            rows = r0 + lane
            x = plsc.load_gather(rows_v, [rows, czero])
            y = plsc.load_gather(rows_v, [rows, czero + 1])
            z = plsc.load_gather(rows_v, [rows, czero + 2])
            pla_v[0, pl.ds(r0, 16)] = x
            pla_v[1, pl.ds(r0, 16)] = y
            pla_v[2, pl.ds(r0, 16)] = z
            pla_v[3, pl.ds(r0, 16)] = fone
            pla_v[4, pl.ds(r0, 16)] = x * x + y * y + z * z
            pla_v[5, pl.ds(r0, 16)] = fzero
            pla_v[6, pl.ds(r0, 16)] = fzero
            pla_v[7, pl.ds(r0, 16)] = fzero
            return carry

        lax.fori_loop(0, PER_W // 16, body, 0)

        side = wid // 16
        bat = (wid // 2) % B
        half = wid % 2
        pltpu.sync_copy(
            pla_v, out_hbm.at[side, bat, :, pl.ds(half * PER_W, PER_W)])

    return gather_kernel(table, idx)


def _dense_body(ga1, ga2, gf1, gf2, cm1, cm2, out, seg):
    b = pl.program_id(0)
    t = pl.program_id(1)

    @pl.when((b == 0) & (t == 0))
    def _build_seg():
        jr = lax.broadcasted_iota(jnp.int32, (NRP, R), 0) // MV
        hc = lax.broadcasted_iota(jnp.int32, (NRP, R), 1)
        seg[...] = (jr == hc).astype(jnp.float32)

    @pl.when(t == 0)
    def _init():
        out[...] = jnp.zeros_like(out)

    def half(a_ref, b_ref, cm):
        p = a_ref[0][0]
        aaug = jnp.concatenate(
            [-2.0 * p[:3, :], p[4:5, :], p[3:4, :],
             jnp.zeros((KA - 5, RT), jnp.float32)], axis=0)
        d2 = lax.dot_general(
            aaug, b_ref[0][0], (((0,), (0,)), ((), ())),
            preferred_element_type=jnp.float32)
        d2 = jnp.maximum(d2, 1e-12)
        cmin = jnp.min(d2.reshape(NRG, MV, NRP), axis=1)
        s = jnp.dot(cmin, seg[...], preferred_element_type=jnp.float32)
        mask = (cm[0] != 0.0).astype(jnp.float32)
        return jnp.sum(s * mask)

    contrib = half(ga1, gf2, cm1) + half(ga2, gf1, cm2)
    out[...] += contrib * (1.0 / MV)


def _dense_call(g, cm1, cm2):
    return pl.pallas_call(
        _dense_body,
        grid=(B, NT),
        in_specs=[
            pl.BlockSpec((1, 1, KA, RT), lambda b, t: (0, b, 0, t)),
            pl.BlockSpec((1, 1, KA, RT), lambda b, t: (1, b, 0, t)),
            pl.BlockSpec((1, 1, KA, NRP), lambda b, t: (0, b, 0, 0)),
            pl.BlockSpec((1, 1, KA, NRP), lambda b, t: (1, b, 0, 0)),
            pl.BlockSpec((1, NRG, R), lambda b, t: (b * NT + t, 0, 0)),
            pl.BlockSpec((1, NRG, R), lambda b, t: (b * NT + t, 0, 0)),
        ],
        out_specs=pl.BlockSpec((1, 1, 128), lambda b, t: (b, 0, 0)),
        out_shape=jax.ShapeDtypeStruct((B, 1, 128), jnp.float32),
        scratch_shapes=[
            pltpu.VMEM((NRP, R), jnp.float32),
        ],
        compiler_params=pltpu.CompilerParams(
            dimension_semantics=("arbitrary", "arbitrary")),
    )(g, g, g, g, cm1, cm2)


def kernel(v1, v2, cmap, rid_to_vid_list):
    f32 = jnp.float32
    v1 = v1.astype(f32)
    v2 = v2.astype(f32)

    t = jnp.concatenate([v1.reshape(B * NV, 3), v2.reshape(B * NV, 3)], axis=0)
    t = jnp.pad(t, ((0, 0), (0, KF - 3)))

    rid = rid_to_vid_list.reshape(-1).astype(jnp.int32)
    rid = jnp.pad(rid, (0, NRP - NR))
    boff = (jnp.arange(B, dtype=jnp.int32) * NV)[None, :, None]
    soff = (jnp.arange(2, dtype=jnp.int32) * (B * NV))[:, None, None]
    idx = (rid[None, None, :] + boff + soff).reshape(-1)

    g = _sc_gather(t, idx)

    cmf = cmap.astype(f32)
    cm1 = jnp.pad(cmf, ((0, 0), (0, RP - R), (0, 0))).reshape(B * NT, NRG, R)
    cm2 = jnp.pad(cmf.transpose(0, 2, 1),
                  ((0, 0), (0, RP - R), (0, 0))).reshape(B * NT, NRG, R)

    out = _dense_call(g, cm1, cm2)
    return out[:, 0, 0]


# speedup vs baseline: 1.2468x; 1.0306x over previous
import functools

import jax
import jax.numpy as jnp
from jax import lax
from jax.experimental import pallas as pl
from jax.experimental.pallas import tpu as pltpu
from jax.experimental.pallas import tpu_sc as plsc

B = 8
NV = 6890
R = 75
RP = 80
MV = 40
NR = R * MV
NRP = 3200
KF = 8
KA = 8

NW = 32
PER_W = NRP // 2
TOT = NW * PER_W

NRG = 16
RT = NRG * MV
NT = RP // NRG


VT = 3456
NVT = NW * VT
WPT = VT * 3


def _sc_build_table(vflat):
    mesh = plsc.VectorSubcoreMesh(core_axis_name="c", subcore_axis_name="s")

    @functools.partial(
        pl.kernel,
        out_type=jax.ShapeDtypeStruct((NVT, KF), jnp.float32),
        mesh=mesh,
        scratch_types=[
            pltpu.VMEM((WPT,), jnp.float32),
            pltpu.VMEM((VT, KF), jnp.float32),
        ],
        compiler_params=pltpu.CompilerParams(
            use_tc_tiling_on_sc=False, needs_layout_passes=False),
    )
    def table_kernel(vflat_hbm, table_hbm, wbuf, tbuf):
        wid = lax.axis_index("s") * 2 + lax.axis_index("c")
        pltpu.sync_copy(vflat_hbm.at[pl.ds(wid * WPT, WPT)], wbuf)
        lane = lax.iota(jnp.int32, 16)
        czero = jnp.zeros((16,), jnp.int32)
        fzero = jnp.zeros((16,), jnp.float32)

        def body(i, carry):
            rows = i * 16 + lane
            w0 = rows * 3
            for c in range(3):
                v = plsc.load_gather(wbuf, [w0 + c])
                plsc.store_scatter(tbuf, [rows, czero + c], v)
            for c in range(3, KF):
                plsc.store_scatter(tbuf, [rows, czero + c], fzero)
            return carry

        lax.fori_loop(0, VT // 16, body, 0)
        pltpu.sync_copy(tbuf, table_hbm.at[pl.ds(wid * VT, VT)])

    return table_kernel(vflat)


def _sc_gather(table, idx):
    mesh = plsc.VectorSubcoreMesh(core_axis_name="c", subcore_axis_name="s")

    @functools.partial(
        pl.kernel,
        out_type=jax.ShapeDtypeStruct((2, B, KA, NRP), jnp.float32),
        mesh=mesh,
        scratch_types=[
            pltpu.VMEM((PER_W,), jnp.int32),
            pltpu.VMEM((PER_W, KF), jnp.float32),
            pltpu.VMEM((KA, PER_W), jnp.float32),
            pltpu.SemaphoreType.DMA,
        ],
        compiler_params=pltpu.CompilerParams(
            use_tc_tiling_on_sc=False, needs_layout_passes=False),
    )
    def gather_kernel(table_hbm, idx_hbm, out_hbm, idx_v, rows_v, pla_v, sem):
        wid = lax.axis_index("s") * 2 + lax.axis_index("c")
        base = wid * PER_W
        pltpu.sync_copy(idx_hbm.at[pl.ds(base, PER_W)], idx_v)
        chunks = [(j, 128) for j in range(0, PER_W - 64, 128)]
        chunks.append((PER_W - 64, 64))
        for j, c in chunks:
            pltpu.async_copy(
                table_hbm.at[idx_v.at[pl.ds(j, c)]],
                rows_v.at[pl.ds(j, c)],
                sem,
            )
        for j, c in chunks:
            pltpu.make_async_copy(
                table_hbm.at[idx_v.at[pl.ds(j, c)]],
                rows_v.at[pl.ds(j, c)],
                sem,
            ).wait()

        lane = lax.iota(jnp.int32, 16)
        czero = jnp.zeros((16,), jnp.int32)
        fone = jnp.ones((16,), jnp.float32)
        fzero = jnp.zeros((16,), jnp.float32)

        def body(i, carry):
            r0 = i * 16
            rows = r0 + lane
            x = plsc.load_gather(rows_v, [rows, czero])
            y = plsc.load_gather(rows_v, [rows, czero + 1])
            z = plsc.load_gather(rows_v, [rows, czero + 2])
            pla_v[0, pl.ds(r0, 16)] = x
            pla_v[1, pl.ds(r0, 16)] = y
            pla_v[2, pl.ds(r0, 16)] = z
            pla_v[3, pl.ds(r0, 16)] = fone
            pla_v[4, pl.ds(r0, 16)] = x * x + y * y + z * z
            pla_v[5, pl.ds(r0, 16)] = fzero
            pla_v[6, pl.ds(r0, 16)] = fzero
            pla_v[7, pl.ds(r0, 16)] = fzero
            return carry

        lax.fori_loop(0, PER_W // 16, body, 0)

        side = wid // 16
        bat = (wid // 2) % B
        half = wid % 2
        pltpu.sync_copy(
            pla_v, out_hbm.at[side, bat, :, pl.ds(half * PER_W, PER_W)])

    return gather_kernel(table, idx)


def _dense_body(ga1, ga2, gf1, gf2, cm1, cm2, out, seg):
    b = pl.program_id(0)
    t = pl.program_id(1)

    @pl.when((b == 0) & (t == 0))
    def _build_seg():
        jr = lax.broadcasted_iota(jnp.int32, (NRP, R), 0) // MV
        hc = lax.broadcasted_iota(jnp.int32, (NRP, R), 1)
        seg[...] = (jr == hc).astype(jnp.float32)

    @pl.when(t == 0)
    def _init():
        out[...] = jnp.zeros_like(out)

    def half(a_ref, b_ref, cm):
        p = a_ref[0][0]
        aaug = jnp.concatenate(
            [-2.0 * p[:3, :], p[4:5, :], p[3:4, :],
             jnp.zeros((KA - 5, RT), jnp.float32)], axis=0)
        d2 = lax.dot_general(
            aaug, b_ref[0][0], (((0,), (0,)), ((), ())),
            preferred_element_type=jnp.float32)
        d2 = jnp.maximum(d2, 1e-12)
        cmin = jnp.min(d2.reshape(NRG, MV, NRP), axis=1)
        s = jnp.dot(cmin, seg[...], preferred_element_type=jnp.float32)
        mask = (cm[0] != 0.0).astype(jnp.float32)
        return jnp.sum(s * mask)

    contrib = half(ga1, gf2, cm1) + half(ga2, gf1, cm2)
    out[...] += contrib * (1.0 / MV)


def _dense_call(g, cm1, cm2):
    return pl.pallas_call(
        _dense_body,
        grid=(B, NT),
        in_specs=[
            pl.BlockSpec((1, 1, KA, RT), lambda b, t: (0, b, 0, t)),
            pl.BlockSpec((1, 1, KA, RT), lambda b, t: (1, b, 0, t)),
            pl.BlockSpec((1, 1, KA, NRP), lambda b, t: (0, b, 0, 0)),
            pl.BlockSpec((1, 1, KA, NRP), lambda b, t: (1, b, 0, 0)),
            pl.BlockSpec((1, NRG, R), lambda b, t: (b * NT + t, 0, 0)),
            pl.BlockSpec((1, NRG, R), lambda b, t: (b * NT + t, 0, 0)),
        ],
        out_specs=pl.BlockSpec((1, 1, 128), lambda b, t: (b, 0, 0)),
        out_shape=jax.ShapeDtypeStruct((B, 1, 128), jnp.float32),
        scratch_shapes=[
            pltpu.VMEM((NRP, R), jnp.float32),
        ],
        compiler_params=pltpu.CompilerParams(
            dimension_semantics=("arbitrary", "arbitrary")),
    )(g, g, g, g, cm1, cm2)


def kernel(v1, v2, cmap, rid_to_vid_list):
    f32 = jnp.float32
    v1 = v1.astype(f32)
    v2 = v2.astype(f32)

    vflat = jnp.concatenate([v1.reshape(-1), v2.reshape(-1)])
    vflat = jnp.pad(vflat, (0, NVT * 3 - vflat.shape[0]))
    t = _sc_build_table(vflat)

    rid = rid_to_vid_list.reshape(-1).astype(jnp.int32)
    rid = jnp.pad(rid, (0, NRP - NR))
    boff = (jnp.arange(B, dtype=jnp.int32) * NV)[None, :, None]
    soff = (jnp.arange(2, dtype=jnp.int32) * (B * NV))[:, None, None]
    idx = (rid[None, None, :] + boff + soff).reshape(-1)

    g = _sc_gather(t, idx)

    cmf = cmap.astype(f32)
    cm1 = jnp.pad(cmf, ((0, 0), (0, RP - R), (0, 0))).reshape(B * NT, NRG, R)
    cm2 = jnp.pad(cmf.transpose(0, 2, 1),
                  ((0, 0), (0, RP - R), (0, 0))).reshape(B * NT, NRG, R)

    out = _dense_call(g, cm1, cm2)
    return out[:, 0, 0]


# speedup vs baseline: 1.2816x; 1.0280x over previous
import functools

import jax
import jax.numpy as jnp
from jax import lax
from jax.experimental import pallas as pl
from jax.experimental.pallas import tpu as pltpu
from jax.experimental.pallas import tpu_sc as plsc

B = 8
NV = 6890
R = 75
RP = 80
MV = 40
NR = R * MV
NRP = 3200
KF = 8
KA = 8

NW = 32
PER_W = NRP // 2
TOT = NW * PER_W

NRG = 16
RT = NRG * MV
NT = RP // NRG


VT = 3456
NVT = NW * VT
WPT = VT * 3


def _sc_build_table(vflat):
    mesh = plsc.VectorSubcoreMesh(core_axis_name="c", subcore_axis_name="s")

    @functools.partial(
        pl.kernel,
        out_type=jax.ShapeDtypeStruct((NVT, KF), jnp.float32),
        mesh=mesh,
        scratch_types=[
            pltpu.VMEM((WPT,), jnp.float32),
            pltpu.VMEM((VT, KF), jnp.float32),
        ],
        compiler_params=pltpu.CompilerParams(
            use_tc_tiling_on_sc=False, needs_layout_passes=False),
    )
    def table_kernel(vflat_hbm, table_hbm, wbuf, tbuf):
        wid = lax.axis_index("s") * 2 + lax.axis_index("c")
        pltpu.sync_copy(vflat_hbm.at[pl.ds(wid * WPT, WPT)], wbuf)
        lane = lax.iota(jnp.int32, 16)
        czero = jnp.zeros((16,), jnp.int32)
        fzero = jnp.zeros((16,), jnp.float32)

        def body(i, carry):
            rows = i * 16 + lane
            w0 = rows * 3
            for c in range(3):
                v = plsc.load_gather(wbuf, [w0 + c])
                plsc.store_scatter(tbuf, [rows, czero + c], v)
            for c in range(3, KF):
                plsc.store_scatter(tbuf, [rows, czero + c], fzero)
            return carry

        lax.fori_loop(0, VT // 16, body, 0)
        pltpu.sync_copy(tbuf, table_hbm.at[pl.ds(wid * VT, VT)])

    return table_kernel(vflat)


def _sc_gather(table, idx):
    mesh = plsc.VectorSubcoreMesh(core_axis_name="c", subcore_axis_name="s")

    @functools.partial(
        pl.kernel,
        out_type=jax.ShapeDtypeStruct((2, B, KA, NRP), jnp.float32),
        mesh=mesh,
        scratch_types=[
            pltpu.VMEM((PER_W,), jnp.int32),
            pltpu.VMEM((PER_W, KF), jnp.float32),
            pltpu.VMEM((KA, PER_W), jnp.float32),
            pltpu.SemaphoreType.DMA,
        ],
        compiler_params=pltpu.CompilerParams(
            use_tc_tiling_on_sc=False, needs_layout_passes=False),
    )
    def gather_kernel(table_hbm, idx_hbm, out_hbm, idx_v, rows_v, pla_v, sem):
        wid = lax.axis_index("s") * 2 + lax.axis_index("c")
        base = wid * PER_W
        pltpu.sync_copy(idx_hbm.at[pl.ds(base, PER_W)], idx_v)
        chunks = [(j, 128) for j in range(0, PER_W - 64, 128)]
        chunks.append((PER_W - 64, 64))
        for j, c in chunks:
            pltpu.async_copy(
                table_hbm.at[idx_v.at[pl.ds(j, c)]],
                rows_v.at[pl.ds(j, c)],
                sem,
            )
        for j, c in chunks:
            pltpu.make_async_copy(
                table_hbm.at[idx_v.at[pl.ds(j, c)]],
                rows_v.at[pl.ds(j, c)],
                sem,
            ).wait()

        lane = lax.iota(jnp.int32, 16)
        czero = jnp.zeros((16,), jnp.int32)
        fone = jnp.ones((16,), jnp.float32)
        fzero = jnp.zeros((16,), jnp.float32)

        def body(i, carry):
            r0 = i * 16
            rows = r0 + lane
            x = plsc.load_gather(rows_v, [rows, czero])
            y = plsc.load_gather(rows_v, [rows, czero + 1])
            z = plsc.load_gather(rows_v, [rows, czero + 2])
            pla_v[0, pl.ds(r0, 16)] = x
            pla_v[1, pl.ds(r0, 16)] = y
            pla_v[2, pl.ds(r0, 16)] = z
            pla_v[3, pl.ds(r0, 16)] = fone
            pla_v[4, pl.ds(r0, 16)] = x * x + y * y + z * z
            pla_v[5, pl.ds(r0, 16)] = fzero
            pla_v[6, pl.ds(r0, 16)] = fzero
            pla_v[7, pl.ds(r0, 16)] = fzero
            return carry

        lax.fori_loop(0, PER_W // 16, body, 0)

        side = wid // 16
        bat = (wid // 2) % B
        half = wid % 2
        pltpu.sync_copy(
            pla_v, out_hbm.at[side, bat, :, pl.ds(half * PER_W, PER_W)])

    return gather_kernel(table, idx)


def _dense_body(ga1, ga2, gf1, gf2, cm1, cm2, out, seg):
    b = pl.program_id(0)
    t = pl.program_id(1)

    @pl.when((b == 0) & (t == 0))
    def _build_seg():
        jr = lax.broadcasted_iota(jnp.int32, (NRP, R), 0) // MV
        hc = lax.broadcasted_iota(jnp.int32, (NRP, R), 1)
        seg[...] = (jr == hc).astype(jnp.float32)

    @pl.when(t == 0)
    def _init():
        out[...] = jnp.zeros_like(out)

    def half(a_ref, b_ref, cm):
        p = a_ref[0][0]
        aaug = jnp.concatenate(
            [-2.0 * p[:3, :], p[4:5, :], p[3:4, :],
             jnp.zeros((KA - 5, RT), jnp.float32)], axis=0)
        d2 = lax.dot_general(
            aaug, b_ref[0][0], (((0,), (0,)), ((), ())),
            preferred_element_type=jnp.float32)
        d2 = jnp.maximum(d2, 1e-12)
        cmin = jnp.min(d2.reshape(NRG, MV, NRP), axis=1)
        s = jnp.dot(cmin, seg[...], preferred_element_type=jnp.float32)
        mask = (cm[0] != 0.0).astype(jnp.float32)
        return jnp.sum(s * mask)

    contrib = half(ga1, gf2, cm1) + half(ga2, gf1, cm2)
    out[...] += contrib * (1.0 / MV)


def _dense_call(g, cm1, cm2):
    return pl.pallas_call(
        _dense_body,
        grid=(B, NT),
        in_specs=[
            pl.BlockSpec((1, 1, KA, RT), lambda b, t: (0, b, 0, t)),
            pl.BlockSpec((1, 1, KA, RT), lambda b, t: (1, b, 0, t)),
            pl.BlockSpec((1, 1, KA, NRP), lambda b, t: (0, b, 0, 0)),
            pl.BlockSpec((1, 1, KA, NRP), lambda b, t: (1, b, 0, 0)),
            pl.BlockSpec((1, NRG, R), lambda b, t: (b * NT + t, 0, 0)),
            pl.BlockSpec((1, NRG, R), lambda b, t: (b * NT + t, 0, 0)),
        ],
        out_specs=pl.BlockSpec((1, 1, 128), lambda b, t: (b, 0, 0)),
        out_shape=jax.ShapeDtypeStruct((B, 1, 128), jnp.float32),
        scratch_shapes=[
            pltpu.VMEM((NRP, R), jnp.float32),
        ],
        compiler_params=pltpu.CompilerParams(
            dimension_semantics=("arbitrary", "arbitrary")),
    )(g, g, g, g, cm1, cm2)


def kernel(v1, v2, cmap, rid_to_vid_list):
    f32 = jnp.float32
    v1 = v1.astype(f32)
    v2 = v2.astype(f32)

    vflat = jnp.stack([v1, v2]).reshape(-1)
    vflat = jnp.pad(vflat, (0, NVT * 3 - vflat.shape[0]))
    t = _sc_build_table(vflat)

    rid = rid_to_vid_list.reshape(-1).astype(jnp.int32)
    rid = jnp.pad(rid, (0, NRP - NR))
    boff = (jnp.arange(B, dtype=jnp.int32) * NV)[None, :, None]
    soff = (jnp.arange(2, dtype=jnp.int32) * (B * NV))[:, None, None]
    idx = (rid[None, None, :] + boff + soff).reshape(-1)

    g = _sc_gather(t, idx)

    cmf = cmap.astype(f32)
    cm1 = jnp.pad(cmf, ((0, 0), (0, RP - R), (0, 0))).reshape(B * NT, NRG, R)
    cm2 = jnp.pad(cmf.transpose(0, 2, 1),
                  ((0, 0), (0, RP - R), (0, 0))).reshape(B * NT, NRG, R)

    out = _dense_call(g, cm1, cm2)
    return out[:, 0, 0]
